# Initial kernel scaffold; baseline (speedup 1.0000x reference)
import functools

import jax
import jax.numpy as jnp
from jax import lax
from jax.experimental import pallas as pl
from jax.experimental.pallas import tpu as pltpu
from jax.experimental.pallas import tpu_sc as plsc

_VOCAB = 100000
_EMBED_DIM = 64
_BATCH = 4096
_SEQ_LEN = 200

_TOTAL = _BATCH * _SEQ_LEN
_NUM_WORKERS = 32
_PER_WORKER = _TOTAL // _NUM_WORKERS
_CHUNK = 128
_NCHUNK = _PER_WORKER // _CHUNK

_mesh = plsc.VectorSubcoreMesh(core_axis_name="c", subcore_axis_name="s")


@functools.partial(
    pl.kernel,
    mesh=_mesh,
    out_type=jax.ShapeDtypeStruct((_TOTAL, _EMBED_DIM), jnp.float32),
    scratch_types=[
        pltpu.VMEM((_NCHUNK, _CHUNK), jnp.int32),
        pltpu.VMEM((_CHUNK, _EMBED_DIM), jnp.float32),
        pltpu.SemaphoreType.DMA,
    ],
)
def _embedding_gather(idx_hbm, table_hbm, out_hbm, idx_v, rows_v, sem):
    wid = lax.axis_index("s") * 2 + lax.axis_index("c")
    pltpu.sync_copy(idx_hbm.at[pl.ds(wid * _NCHUNK, _NCHUNK)], idx_v)

    def body(j, carry):
        pltpu.async_copy(table_hbm.at[idx_v.at[j]], rows_v, sem).wait()
        base = wid * _PER_WORKER + j * _CHUNK
        pltpu.sync_copy(rows_v, out_hbm.at[pl.ds(base, _CHUNK)])
        return carry

    lax.fori_loop(0, _NCHUNK, body, 0)


def kernel(indices, embedding_matrix):
    idx = indices.reshape(_TOTAL // _CHUNK, _CHUNK).astype(jnp.int32)
    out = _embedding_gather(idx, embedding_matrix)
    return out.reshape(_BATCH, _SEQ_LEN, _EMBED_DIM)


# speedup vs baseline: 3.5503x; 3.5503x over previous
import functools

import jax
import jax.numpy as jnp
from jax import lax
from jax.experimental import pallas as pl
from jax.experimental.pallas import tpu as pltpu
from jax.experimental.pallas import tpu_sc as plsc

_VOCAB = 100000
_EMBED_DIM = 64
_BATCH = 4096
_SEQ_LEN = 200

_TOTAL = _BATCH * _SEQ_LEN
_NUM_WORKERS = 32
_PER_WORKER = _TOTAL // _NUM_WORKERS
_CHUNK = 128
_NCHUNK = _PER_WORKER // _CHUNK

_mesh = plsc.VectorSubcoreMesh(core_axis_name="c", subcore_axis_name="s")


@functools.partial(
    pl.kernel,
    mesh=_mesh,
    out_type=jax.ShapeDtypeStruct((_TOTAL, _EMBED_DIM), jnp.float32),
    scratch_types=[
        pltpu.VMEM((_NCHUNK, _CHUNK), jnp.int32),
        pltpu.VMEM((_CHUNK, _EMBED_DIM), jnp.float32),
        pltpu.SemaphoreType.DMA,
    ],
    compiler_params=pltpu.CompilerParams(use_tc_tiling_on_sc=False),
)
def _embedding_gather(idx_hbm, table_hbm, out_hbm, idx_v, rows_v, sem):
    wid = lax.axis_index("s") * 2 + lax.axis_index("c")
    pltpu.sync_copy(idx_hbm.at[pl.ds(wid * _NCHUNK, _NCHUNK)], idx_v)

    def body(j, carry):
        pltpu.async_copy(table_hbm.at[idx_v.at[j]], rows_v, sem).wait()
        base = wid * _PER_WORKER + j * _CHUNK
        pltpu.sync_copy(rows_v, out_hbm.at[pl.ds(base, _CHUNK)])
        return carry

    lax.fori_loop(0, _NCHUNK, body, 0)


def kernel(indices, embedding_matrix):
    idx = indices.reshape(_TOTAL // _CHUNK, _CHUNK).astype(jnp.int32)
    out = _embedding_gather(idx, embedding_matrix)
    return out.reshape(_BATCH, _SEQ_LEN, _EMBED_DIM)


# speedup vs baseline: 3.7715x; 1.0623x over previous
import functools

import jax
import jax.numpy as jnp
from jax import lax
from jax.experimental import pallas as pl
from jax.experimental.pallas import tpu as pltpu
from jax.experimental.pallas import tpu_sc as plsc

_VOCAB = 100000
_EMBED_DIM = 64
_BATCH = 4096
_SEQ_LEN = 200

_TOTAL = _BATCH * _SEQ_LEN
_NUM_WORKERS = 32
_PER_WORKER = _TOTAL // _NUM_WORKERS
_CHUNK = 128
_NCHUNK = _PER_WORKER // _CHUNK

_mesh = plsc.VectorSubcoreMesh(core_axis_name="c", subcore_axis_name="s")


@functools.partial(
    pl.kernel,
    mesh=_mesh,
    out_type=jax.ShapeDtypeStruct((_TOTAL, _EMBED_DIM), jnp.float32),
    scratch_types=[
        pltpu.VMEM((_NCHUNK, _CHUNK), jnp.int32),
        pltpu.VMEM((_CHUNK, _EMBED_DIM), jnp.float32),
        pltpu.VMEM((_CHUNK, _EMBED_DIM), jnp.float32),
        pltpu.SemaphoreType.DMA,
        pltpu.SemaphoreType.DMA,
    ],
    compiler_params=pltpu.CompilerParams(use_tc_tiling_on_sc=False),
)
def _embedding_gather(idx_hbm, table_hbm, out_hbm, idx_v, rows0, rows1, sem0, sem1):
    wid = lax.axis_index("s") * 2 + lax.axis_index("c")
    pltpu.sync_copy(idx_hbm.at[pl.ds(wid * _NCHUNK, _NCHUNK)], idx_v)
    base = wid * _PER_WORKER

    pltpu.async_copy(table_hbm.at[idx_v.at[0]], rows0, sem0)

    def body(i, carry):
        j = 2 * i
        pltpu.make_async_copy(table_hbm.at[idx_v.at[j]], rows0, sem0).wait()
        pltpu.async_copy(table_hbm.at[idx_v.at[j + 1]], rows1, sem1)
        pltpu.sync_copy(rows0, out_hbm.at[pl.ds(base + j * _CHUNK, _CHUNK)])

        pltpu.make_async_copy(table_hbm.at[idx_v.at[j + 1]], rows1, sem1).wait()

        @pl.when(j + 2 < _NCHUNK)
        def _():
            pltpu.async_copy(table_hbm.at[idx_v.at[j + 2]], rows0, sem0)

        pltpu.sync_copy(rows1, out_hbm.at[pl.ds(base + (j + 1) * _CHUNK, _CHUNK)])
        return carry

    lax.fori_loop(0, _NCHUNK // 2, body, 0)


def kernel(indices, embedding_matrix):
    idx = indices.reshape(_TOTAL // _CHUNK, _CHUNK).astype(jnp.int32)
    out = _embedding_gather(idx, embedding_matrix)
    return out.reshape(_BATCH, _SEQ_LEN, _EMBED_DIM)
